# baseline (speedup 1.0000x reference)
import functools

import jax
import jax.numpy as jnp
from jax import lax
from jax.experimental import pallas as pl
from jax.experimental.pallas import tpu as pltpu
from jax.experimental.pallas import tpu_sc as plsc

N = 10000
D = 128
G = 16
NHOPS = 3
E = 320000
NCORES = 2
NTILES = 16
NW = NCORES * NTILES
CHUNK = 128
NBUF = 2
CHUNKS = NBUF * (-(-E // (NW * CHUNK * NBUF)))
TAIL = 8
IDXC = CHUNKS + TAIL
NPASS = 2
PCHUNKS = CHUNKS // NPASS
PIDX = PCHUNKS + TAIL
EPW = CHUNKS * CHUNK
PADDED_E = NW * EPW
PAD_ROWS = 112
NPAD = N + PAD_ROWS
ZROWS = NPAD // NTILES
RBLK = 1000
NBLK = N // RBLK


def _mlp4_body(ins, w1_ref, b1_ref, w2_ref, b2_ref, acc):
    for i in range(NHOPS + 1):
        h = jnp.dot(ins[i], w1_ref[i], preferred_element_type=jnp.float32)
        h = jnp.maximum(h + b1_ref[i][None, :], 0.0)
        acc = acc + jnp.dot(h, w2_ref[i], preferred_element_type=jnp.float32)
        acc = acc + b2_ref[i][None, :]
    return acc


def _x1_kernel(f_ref, h1_ref, h2_ref, h3_ref, w1_ref, b1_ref, w2_ref, b2_ref,
               out_ref):
    ins = (f_ref[...], h1_ref[...], h2_ref[...], h3_ref[...])
    out_ref[...] = _mlp4_body(ins, w1_ref, b1_ref, w2_ref, b2_ref,
                              jnp.zeros_like(out_ref))


def _row_spec():
    return pl.BlockSpec((RBLK, D), lambda i: (i, 0))


def _full(shape):
    return pl.BlockSpec(shape, lambda i: (0,) * len(shape))


def _x1_call(f, h1, h2, h3, w1, b1, w2, b2):
    return pl.pallas_call(
        _x1_kernel,
        grid=(NBLK,),
        in_specs=[_row_spec()] * 4 + [
            _full((NHOPS + 1, D, D)), _full((NHOPS + 1, D)),
            _full((NHOPS + 1, D, D)), _full((NHOPS + 1, D)),
        ],
        out_specs=_row_spec(),
        out_shape=jax.ShapeDtypeStruct((N, D), jnp.float32),
    )(f, h1, h2, h3, w1, b1, w2, b2)


def _sc_agg(x1_hbm, srcs_hbm, dsts_hbm, zeros_hbm, out_hbm,
            src_v, dst_v, r0, r1, acc_sh, g0, g1):
    rows = (r0, r1)
    gsems = (g0, g1)
    c = lax.axis_index("c")
    s = lax.axis_index("s")
    zsl = pl.ds(s * ZROWS, ZROWS)
    pltpu.sync_copy(zeros_hbm.at[zsl], acc_sh.at[zsl])
    plsc.subcore_barrier()
    for p in range(NPASS):
        pltpu.sync_copy(srcs_hbm.at[c, s, pl.ds(p * PCHUNKS, PIDX)], src_v)
        pltpu.sync_copy(dsts_hbm.at[c, s, pl.ds(p * PCHUNKS, PIDX)], dst_v)

        def _gather_start(j, b):
            pltpu.async_copy(x1_hbm.at[src_v.at[j]], rows[b], gsems[b])

        def _gather_wait(b):
            pltpu.make_async_copy(zeros_hbm.at[pl.ds(0, CHUNK)], rows[b],
                                  gsems[b]).wait()

        for b in range(NBUF):
            _gather_start(b, b)

        @pl.loop(0, PCHUNKS // NBUF)
        def _chunk_loop(jj):
            base = jj * NBUF
            for b in range(NBUF):
                _gather_wait(b)
                pltpu.sync_copy(rows[b], acc_sh.at[dst_v.at[base + b]],
                                add=True)
                _gather_start(base + NBUF + b, b)

        for b in range(NBUF):
            _gather_wait(b)
    plsc.subcore_barrier()
    pltpu.sync_copy(acc_sh.at[zsl], out_hbm.at[c, zsl])


def _sc_hop_call(x1, srcs_h, dsts_h, zeros):
    fn = functools.partial(
        pl.kernel, _sc_agg,
        out_type=jax.ShapeDtypeStruct((NCORES, NPAD, D), jnp.float32),
        mesh=plsc.VectorSubcoreMesh(core_axis_name="c", subcore_axis_name="s"),
        scratch_types=[
            pltpu.VMEM((PIDX, CHUNK), jnp.int32),
            pltpu.VMEM((PIDX, CHUNK), jnp.int32),
            pltpu.VMEM((CHUNK, D), jnp.float32),
            pltpu.VMEM((CHUNK, D), jnp.float32),
            pltpu.VMEM_SHARED((NPAD, D), jnp.float32),
            pltpu.SemaphoreType.DMA,
            pltpu.SemaphoreType.DMA,
        ],
    )()
    return fn(x1, srcs_h, dsts_h, zeros)


def _prep_edges(e1, e2, e3):
    pad = PADDED_E - E
    ar = jnp.arange(pad, dtype=jnp.int32)
    dpad = TAIL * CHUNK
    dar = jnp.arange(dpad, dtype=jnp.int32)
    dsrc = jnp.broadcast_to((dar % N).reshape(1, 1, TAIL, CHUNK),
                            (NCORES, NTILES, TAIL, CHUNK))
    ddst = jnp.broadcast_to((N + dar % PAD_ROWS).reshape(1, 1, TAIL, CHUNK),
                            (NCORES, NTILES, TAIL, CHUNK))
    srcs, dsts = [], []
    for e in (e1, e2, e3):
        src = jnp.concatenate([e[1], ar % N])
        dst = jnp.concatenate([e[0], N + (ar % PAD_ROWS)])
        src = src.reshape(NCORES, NTILES, CHUNKS, CHUNK)
        dst = dst.reshape(NCORES, NTILES, CHUNKS, CHUNK)
        srcs.append(jnp.concatenate([src, dsrc], axis=2))
        dsts.append(jnp.concatenate([dst, ddst], axis=2))
    return srcs, dsts


def _hop_mlp_kernel(p_ref, w1_ref, b1_ref, w2_ref, b2_ref, out_ref):
    h = p_ref[0] + p_ref[1]
    h = jnp.dot(h, w1_ref[...], preferred_element_type=jnp.float32)
    h = jnp.maximum(h + b1_ref[...], 0.0)
    out_ref[...] = (jnp.dot(h, w2_ref[...], preferred_element_type=jnp.float32)
                    + b2_ref[...])


def _hop_mlp_call(part, w1, b1, w2, b2):
    return pl.pallas_call(
        _hop_mlp_kernel,
        grid=(NBLK,),
        in_specs=[
            pl.BlockSpec((NCORES, RBLK, D), lambda i: (0, i, 0)),
            _full((D, D)), _full((1, D)), _full((D, D)), _full((1, D)),
        ],
        out_specs=_row_spec(),
        out_shape=jax.ShapeDtypeStruct((N, D), jnp.float32),
    )(part, w1, b1, w2, b2)


def _final_kernel(x1_ref, y1_ref, y2_ref, y3_ref, w1_ref, b1_ref, w2_ref,
                  b2_ref, linw_ref, linb_ref, batch_ref, out_ref):
    @pl.when(pl.program_id(0) == 0)
    def _init():
        out_ref[...] = jnp.zeros_like(out_ref)

    x1b = x1_ref[...]
    h = jnp.dot(x1b, w1_ref[...], preferred_element_type=jnp.float32)
    h = jnp.maximum(h + b1_ref[...], 0.0)
    x2 = (jnp.dot(h, w2_ref[...], preferred_element_type=jnp.float32)
          + b2_ref[...] + y1_ref[...] + y2_ref[...] + y3_ref[...])
    proj = (jnp.dot(x1b, linw_ref[0], preferred_element_type=jnp.float32)
            + jnp.dot(x2, linw_ref[1], preferred_element_type=jnp.float32)
            + linb_ref[...])
    b = batch_ref[0, 0, :]
    onehot = (b[None, :] == lax.broadcasted_iota(jnp.int32, (G, RBLK), 0)
              ).astype(jnp.float32)
    out_ref[...] += jnp.dot(onehot, proj, preferred_element_type=jnp.float32)


def _final_call(x1, y1, y2, y3, w1, b1, w2, b2, linw, linb, batch3):
    return pl.pallas_call(
        _final_kernel,
        grid=(NBLK,),
        in_specs=[
            _row_spec(), _row_spec(), _row_spec(), _row_spec(),
            _full((D, D)), _full((1, D)), _full((D, D)), _full((1, D)),
            _full((2, D, D)), _full((1, D)),
            pl.BlockSpec((1, 1, RBLK), lambda i: (i, 0, 0)),
        ],
        out_specs=_full((G, D)),
        out_shape=jax.ShapeDtypeStruct((G, D), jnp.float32),
    )(x1, y1, y2, y3, w1, b1, w2, b2, linw, linb, batch3)


def kernel(x, rw_feature, hop1_features, hop2_features, hop3_features,
           hop1_edges, hop2_edges, hop3_edges, batch,
           W1_l1, b1_l1, W2_l1, b2_l1, W1_l2, b1_l2, W2_l2, b2_l2,
           lin_W, lin_b):
    f = jnp.concatenate([x, rw_feature], axis=1)
    x1 = _x1_call(f, hop1_features, hop2_features, hop3_features,
                  W1_l1, b1_l1, W2_l1, b2_l1)
    srcs, dsts = _prep_edges(hop1_edges, hop2_edges, hop3_edges)
    zeros = jnp.zeros((NPAD, D), jnp.float32)
    ys = []
    for h in range(NHOPS):
        part = _sc_hop_call(x1, srcs[h], dsts[h], zeros)
        ys.append(_hop_mlp_call(part, W1_l2[h + 1], b1_l2[h + 1].reshape(1, D),
                                W2_l2[h + 1], b2_l2[h + 1].reshape(1, D)))
    return _final_call(x1, ys[0], ys[1], ys[2],
                       W1_l2[0], b1_l2[0].reshape(1, D),
                       W2_l2[0], b2_l2[0].reshape(1, D),
                       lin_W.reshape(2, D, D), lin_b.reshape(1, D),
                       batch.reshape(NBLK, 1, RBLK))


# speedup vs baseline: 1.0648x; 1.0648x over previous
import functools

import jax
import jax.numpy as jnp
from jax import lax
from jax.experimental import pallas as pl
from jax.experimental.pallas import tpu as pltpu
from jax.experimental.pallas import tpu_sc as plsc

N = 10000
D = 128
G = 16
NHOPS = 3
E = 320000
NCORES = 2
NTILES = 16
NW = NCORES * NTILES
CHUNK = 128
NBUF = 2
CHUNKS = 80
TAIL = 8
IDXC = CHUNKS + TAIL
NPASS = 2
PCHUNKS = CHUNKS // NPASS
PIDX = PCHUNKS + TAIL
EPW = CHUNKS * CHUNK
PADDED_E = NW * EPW
PAD_ROWS = 112
NPAD = N + PAD_ROWS
ZROWS = NPAD // NTILES
RBLK = 2000
NBLK = N // RBLK


def _mlp4_body(ins, w1_ref, b1_ref, w2_ref, b2_ref, acc):
    for i in range(NHOPS + 1):
        h = jnp.dot(ins[i], w1_ref[i], preferred_element_type=jnp.float32)
        h = jnp.maximum(h + b1_ref[i][None, :], 0.0)
        acc = acc + jnp.dot(h, w2_ref[i], preferred_element_type=jnp.float32)
        acc = acc + b2_ref[i][None, :]
    return acc


def _x1_kernel(f_ref, h1_ref, h2_ref, h3_ref, w1_ref, b1_ref, w2_ref, b2_ref,
               out_ref):
    ins = (f_ref[...], h1_ref[...], h2_ref[...], h3_ref[...])
    out_ref[...] = _mlp4_body(ins, w1_ref, b1_ref, w2_ref, b2_ref,
                              jnp.zeros_like(out_ref))


def _row_spec():
    return pl.BlockSpec((RBLK, D), lambda i: (i, 0))


def _full(shape):
    return pl.BlockSpec(shape, lambda i: (0,) * len(shape))


def _x1_call(f, h1, h2, h3, w1, b1, w2, b2):
    return pl.pallas_call(
        _x1_kernel,
        grid=(NBLK,),
        in_specs=[_row_spec()] * 4 + [
            _full((NHOPS + 1, D, D)), _full((NHOPS + 1, D)),
            _full((NHOPS + 1, D, D)), _full((NHOPS + 1, D)),
        ],
        out_specs=_row_spec(),
        out_shape=jax.ShapeDtypeStruct((N, D), jnp.float32),
    )(f, h1, h2, h3, w1, b1, w2, b2)


def _sc_agg(x1_hbm, srcs_hbm, dsts_hbm, zeros_hbm, out_hbm,
            src_v, dst_v, r0, r1, acc_sh, g0, g1):
    rows = (r0, r1)
    gsems = (g0, g1)
    c = lax.axis_index("c")
    s = lax.axis_index("s")
    zsl = pl.ds(s * ZROWS, ZROWS)
    for hop in range(NHOPS):
        pltpu.sync_copy(zeros_hbm.at[zsl], acc_sh.at[zsl])
        plsc.subcore_barrier()
        for p in range(NPASS):
            pltpu.sync_copy(srcs_hbm.at[hop, c, s, pl.ds(p * PCHUNKS, PIDX)],
                            src_v)
            pltpu.sync_copy(dsts_hbm.at[hop, c, s, pl.ds(p * PCHUNKS, PIDX)],
                            dst_v)

            def _gather_start(j, b):
                pltpu.async_copy(x1_hbm.at[src_v.at[j]], rows[b], gsems[b])

            def _gather_wait(b):
                pltpu.make_async_copy(zeros_hbm.at[pl.ds(0, CHUNK)], rows[b],
                                      gsems[b]).wait()

            for b in range(NBUF):
                _gather_start(b, b)

            @pl.loop(0, PCHUNKS // NBUF)
            def _chunk_loop(jj):
                base = jj * NBUF
                for b in range(NBUF):
                    _gather_wait(b)
                    pltpu.sync_copy(rows[b], acc_sh.at[dst_v.at[base + b]],
                                    add=True)
                    _gather_start(base + NBUF + b, b)

            for b in range(NBUF):
                _gather_wait(b)
        plsc.subcore_barrier()
        pltpu.sync_copy(acc_sh.at[zsl], out_hbm.at[hop, c, zsl])


def _sc_call(x1, srcs, dsts, zeros):
    fn = functools.partial(
        pl.kernel, _sc_agg,
        out_type=jax.ShapeDtypeStruct((NHOPS, NCORES, NPAD, D), jnp.float32),
        mesh=plsc.VectorSubcoreMesh(core_axis_name="c", subcore_axis_name="s"),
        scratch_types=[
            pltpu.VMEM((PIDX, CHUNK), jnp.int32),
            pltpu.VMEM((PIDX, CHUNK), jnp.int32),
            pltpu.VMEM((CHUNK, D), jnp.float32),
            pltpu.VMEM((CHUNK, D), jnp.float32),
            pltpu.VMEM_SHARED((NPAD, D), jnp.float32),
            pltpu.SemaphoreType.DMA,
            pltpu.SemaphoreType.DMA,
        ],
    )()
    return fn(x1, srcs, dsts, zeros)


def _prep_edges(e1, e2, e3):
    pad = PADDED_E - E
    ar = jnp.arange(pad, dtype=jnp.int32)
    dpad = TAIL * CHUNK
    dar = jnp.arange(dpad, dtype=jnp.int32)
    dsrc = jnp.broadcast_to((dar % N).reshape(1, 1, TAIL, CHUNK),
                            (NCORES, NTILES, TAIL, CHUNK))
    ddst = jnp.broadcast_to((N + dar % PAD_ROWS).reshape(1, 1, TAIL, CHUNK),
                            (NCORES, NTILES, TAIL, CHUNK))
    srcs, dsts = [], []
    for e in (e1, e2, e3):
        src = jnp.concatenate([e[1], ar % N])
        dst = jnp.concatenate([e[0], N + (ar % PAD_ROWS)])
        src = src.reshape(NCORES, NTILES, CHUNKS, CHUNK)
        dst = dst.reshape(NCORES, NTILES, CHUNKS, CHUNK)
        srcs.append(jnp.concatenate([src, dsrc], axis=2))
        dsts.append(jnp.concatenate([dst, ddst], axis=2))
    return jnp.stack(srcs), jnp.stack(dsts)


def _final_kernel(x1_ref, p_ref, w1_ref, b1_ref, w2_ref, b2_ref,
                  linw_ref, linb_ref, batch_ref, out_ref):
    @pl.when(pl.program_id(0) == 0)
    def _init():
        out_ref[...] = jnp.zeros_like(out_ref)

    x1b = x1_ref[...]
    ins = (x1b,
           p_ref[0, 0] + p_ref[0, 1],
           p_ref[1, 0] + p_ref[1, 1],
           p_ref[2, 0] + p_ref[2, 1])
    x2 = _mlp4_body(ins, w1_ref, b1_ref, w2_ref, b2_ref, jnp.zeros_like(x1b))
    proj = (jnp.dot(x1b, linw_ref[0], preferred_element_type=jnp.float32)
            + jnp.dot(x2, linw_ref[1], preferred_element_type=jnp.float32)
            + linb_ref[...])
    b = batch_ref[0, 0, :]
    onehot = (b[None, :] == lax.broadcasted_iota(jnp.int32, (G, RBLK), 0)
              ).astype(jnp.float32)
    out_ref[...] += jnp.dot(onehot, proj, preferred_element_type=jnp.float32)


def _final_call(x1, parts, w1, b1, w2, b2, linw, linb, batch3):
    return pl.pallas_call(
        _final_kernel,
        grid=(NBLK,),
        in_specs=[
            _row_spec(),
            pl.BlockSpec((NHOPS, NCORES, RBLK, D), lambda i: (0, 0, i, 0)),
            _full((NHOPS + 1, D, D)), _full((NHOPS + 1, D)),
            _full((NHOPS + 1, D, D)), _full((NHOPS + 1, D)),
            _full((2, D, D)), _full((1, D)),
            pl.BlockSpec((1, 1, RBLK), lambda i: (i, 0, 0)),
        ],
        out_specs=_full((G, D)),
        out_shape=jax.ShapeDtypeStruct((G, D), jnp.float32),
    )(x1, parts, w1, b1, w2, b2, linw, linb, batch3)


def kernel(x, rw_feature, hop1_features, hop2_features, hop3_features,
           hop1_edges, hop2_edges, hop3_edges, batch,
           W1_l1, b1_l1, W2_l1, b2_l1, W1_l2, b1_l2, W2_l2, b2_l2,
           lin_W, lin_b):
    f = jnp.concatenate([x, rw_feature], axis=1)
    x1 = _x1_call(f, hop1_features, hop2_features, hop3_features,
                  W1_l1, b1_l1, W2_l1, b2_l1)
    srcs, dsts = _prep_edges(hop1_edges, hop2_edges, hop3_edges)
    zeros = jnp.zeros((NPAD, D), jnp.float32)
    parts = _sc_call(x1, srcs, dsts, zeros)
    return _final_call(x1, parts, W1_l2, b1_l2, W2_l2, b2_l2,
                       lin_W.reshape(2, D, D), lin_b.reshape(1, D),
                       batch.reshape(NBLK, 1, RBLK))
